# Initial kernel scaffold; baseline (speedup 1.0000x reference)
import jax
import jax.numpy as jnp
from jax.experimental import pallas as pl


def kernel(x, t, emb_weight, lin_weight, lin_bias):
    raise NotImplementedError("write your pallas kernel here")


# speedup vs baseline: 1.8055x; 1.8055x over previous
import functools
import jax
import jax.numpy as jnp
from jax import lax
from jax.experimental import pallas as pl
from jax.experimental.pallas import tpu as pltpu
from jax.experimental.pallas import tpu_sc as plsc

_NC = 2
_NS = 16
_NW = _NC * _NS
_L = 16

_CHUNK = 80


def _proj_body(emb_ref, w_ref, b_ref, out_ref):
    out_ref[...] = lax.dot_general(
        emb_ref[...], w_ref[...],
        (((1,), (1,)), ((), ())),
        preferred_element_type=jnp.float32,
    ) + b_ref[...]


def _compute_proj(emb_weight, lin_weight, lin_bias):
    m, d = emb_weight.shape
    return pl.pallas_call(
        _proj_body,
        out_shape=jax.ShapeDtypeStruct((m, d), jnp.float32),
    )(emb_weight, lin_weight, lin_bias.reshape(1, d))


def _sc_gather_add(x, t, proj):
    n, d = x.shape
    per_w = n // _NW
    assert per_w * _NW == n and per_w % _CHUNK == 0
    n_chunks = per_w // _CHUNK
    mesh = plsc.VectorSubcoreMesh(
        core_axis_name="c", subcore_axis_name="s",
        num_cores=_NC, num_subcores=_NS,
    )

    @functools.partial(
        pl.kernel,
        out_type=jax.ShapeDtypeStruct((n, d), jnp.float32),
        mesh=mesh,
        scratch_types=[
            pltpu.VMEM((_CHUNK,), jnp.int32),
            pltpu.VMEM((_CHUNK, d), jnp.float32),
            pltpu.VMEM((_CHUNK, d), jnp.float32),
            pltpu.SemaphoreType.DMA,
            pltpu.SemaphoreType.DMA,
        ],
    )
    def run(x_hbm, t_hbm, proj_hbm, out_hbm, idx_v, x_v, rows_v, xsem, gsem):
        wid = lax.axis_index("s") * _NC + lax.axis_index("c")
        w_base = wid * per_w

        def chunk_body(k, carry):
            base = w_base + k * _CHUNK
            cp_x = pltpu.async_copy(x_hbm.at[pl.ds(base, _CHUNK)], x_v, xsem)
            pltpu.sync_copy(t_hbm.at[pl.ds(base, _CHUNK)], idx_v)
            cp_g = pltpu.async_copy(proj_hbm.at[idx_v], rows_v, gsem)
            cp_x.wait()
            cp_g.wait()

            def tok_body(i, tc):
                for c in range(d // _L):
                    sl = pl.ds(c * _L, _L)
                    plsc.addupdate(x_v.at[i, sl], rows_v[i, sl])
                return tc
            lax.fori_loop(0, _CHUNK, tok_body, 0)

            pltpu.sync_copy(x_v, out_hbm.at[pl.ds(base, _CHUNK)])
            return carry

        lax.fori_loop(0, n_chunks, chunk_body, 0)

    return run(x, t, proj)


def kernel(x, t, emb_weight, lin_weight, lin_bias):
    proj = _compute_proj(emb_weight, lin_weight, lin_bias)
    return _sc_gather_add(x, t, proj)


# speedup vs baseline: 2.1753x; 1.2048x over previous
import functools
import jax
import jax.numpy as jnp
from jax import lax
from jax.experimental import pallas as pl
from jax.experimental.pallas import tpu as pltpu
from jax.experimental.pallas import tpu_sc as plsc

_NC = 2
_NS = 16
_NW = _NC * _NS
_L = 16

_CHUNK = 80
_XR = 3
_GR = 2


def _proj_body(emb_ref, w_ref, b_ref, out_ref):
    out_ref[...] = lax.dot_general(
        emb_ref[...], w_ref[...],
        (((1,), (1,)), ((), ())),
        preferred_element_type=jnp.float32,
    ) + b_ref[...]


def _compute_proj(emb_weight, lin_weight, lin_bias):
    m, d = emb_weight.shape
    return pl.pallas_call(
        _proj_body,
        out_shape=jax.ShapeDtypeStruct((m, d), jnp.float32),
    )(emb_weight, lin_weight, lin_bias.reshape(1, d))


def _sc_gather_add(x, t, proj):
    n, d = x.shape
    per_w = n // _NW
    assert per_w * _NW == n and per_w % _CHUNK == 0
    n_chunks = per_w // _CHUNK
    t3 = t.reshape(_NW, n_chunks, _CHUNK)
    mesh = plsc.VectorSubcoreMesh(
        core_axis_name="c", subcore_axis_name="s",
        num_cores=_NC, num_subcores=_NS,
    )

    @functools.partial(
        pl.kernel,
        out_type=jax.ShapeDtypeStruct((n, d), jnp.float32),
        mesh=mesh,
        scratch_types=[
            pltpu.VMEM((n_chunks, _CHUNK), jnp.int32),
            pltpu.VMEM((_XR, _CHUNK, d), jnp.float32),
            pltpu.VMEM((_GR, _CHUNK, d), jnp.float32),
            pltpu.SemaphoreType.DMA,
            pltpu.SemaphoreType.DMA,
            pltpu.SemaphoreType.DMA,
        ],
    )
    def run(x_hbm, t3_hbm, proj_hbm, out_hbm, idx_all, x_v, rows_v,
            xsem, gsem, osem):
        wid = lax.axis_index("s") * _NC + lax.axis_index("c")
        w_base = wid * per_w

        def x_slice(k):
            return x_hbm.at[pl.ds(w_base + k * _CHUNK, _CHUNK)]

        def out_slice(k):
            return out_hbm.at[pl.ds(w_base + k * _CHUNK, _CHUNK)]

        def issue_inputs(k):
            pltpu.async_copy(x_slice(k), x_v.at[lax.rem(k, _XR)], xsem)
            pltpu.async_copy(proj_hbm.at[idx_all.at[k]],
                             rows_v.at[lax.rem(k, _GR)], gsem)

        pltpu.sync_copy(t3_hbm.at[wid], idx_all)
        issue_inputs(0)
        issue_inputs(1)

        def chunk_body(k, carry):
            xb = lax.rem(k, _XR)
            gb = lax.rem(k, _GR)
            pltpu.make_async_copy(x_slice(k), x_v.at[xb], xsem).wait()
            pltpu.make_async_copy(proj_hbm.at[idx_all.at[k]],
                                  rows_v.at[gb], gsem).wait()

            @plsc.parallel_loop(0, _CHUNK, unroll=4)
            def tok_body(i):
                for c in range(d // _L):
                    sl = pl.ds(c * _L, _L)
                    plsc.addupdate(x_v.at[xb, i, sl], rows_v[gb, i, sl])

            pltpu.async_copy(x_v.at[xb], out_slice(k), osem)

            @pl.when(k + 2 < n_chunks)
            def _():
                @pl.when(k >= 1)
                def _():
                    pltpu.make_async_copy(
                        x_v.at[lax.rem(k - 1, _XR)], out_slice(k - 1), osem
                    ).wait()
                issue_inputs(k + 2)

            return carry

        lax.fori_loop(0, n_chunks, chunk_body, 0)
        for j in range(n_chunks - 3, n_chunks):
            pltpu.make_async_copy(x_v.at[j % _XR], out_slice(j), osem).wait()

    return run(x, t3, proj)


def kernel(x, t, emb_weight, lin_weight, lin_bias):
    proj = _compute_proj(emb_weight, lin_weight, lin_bias)
    return _sc_gather_add(x, t, proj)


# speedup vs baseline: 2.1888x; 1.0062x over previous
import functools
import jax
import jax.numpy as jnp
from jax import lax
from jax.experimental import pallas as pl
from jax.experimental.pallas import tpu as pltpu
from jax.experimental.pallas import tpu_sc as plsc

_NC = 2
_NS = 16
_NW = _NC * _NS
_L = 16

_CHUNK = 80
_XR = 3
_GR = 2


def _proj_body(emb_ref, w_ref, b_ref, out_ref):
    out_ref[...] = lax.dot_general(
        emb_ref[...], w_ref[...],
        (((1,), (1,)), ((), ())),
        preferred_element_type=jnp.float32,
    ) + b_ref[...]


def _compute_proj(emb_weight, lin_weight, lin_bias):
    m, d = emb_weight.shape
    return pl.pallas_call(
        _proj_body,
        out_shape=jax.ShapeDtypeStruct((m, d), jnp.float32),
    )(emb_weight, lin_weight, lin_bias.reshape(1, d))


def _sc_gather_add(x, t, proj):
    n, d = x.shape
    per_w = n // _NW
    assert per_w * _NW == n and per_w % _CHUNK == 0
    n_chunks = per_w // _CHUNK
    t3 = t.reshape(_NW, n_chunks, _CHUNK)
    mesh = plsc.VectorSubcoreMesh(
        core_axis_name="c", subcore_axis_name="s",
        num_cores=_NC, num_subcores=_NS,
    )

    @functools.partial(
        pl.kernel,
        out_type=jax.ShapeDtypeStruct((n, d), jnp.float32),
        mesh=mesh,
        scratch_types=[
            pltpu.VMEM((n_chunks, _CHUNK), jnp.int32),
            pltpu.VMEM((_CHUNK,), jnp.int32),
            pltpu.VMEM_SHARED((_NS, _XR, _CHUNK, d), jnp.float32),
            pltpu.VMEM((_GR, _CHUNK, d), jnp.float32),
            pltpu.SemaphoreType.DMA((_XR,)),
            pltpu.SemaphoreType.DMA((_GR,)),
            pltpu.SemaphoreType.DMA((_XR,)),
        ],
    )
    def run(x_hbm, t3_hbm, proj_hbm, out_hbm, idx_all, idn_v, xs_all, rows_v,
            xsem, gsem, osem):
        sid = lax.axis_index("s")
        wid = sid * _NC + lax.axis_index("c")
        w_base = wid * per_w
        x_v = xs_all.at[sid]

        def x_slice(k):
            return x_hbm.at[pl.ds(w_base + k * _CHUNK, _CHUNK)]

        def out_slice(k):
            return out_hbm.at[pl.ds(w_base + k * _CHUNK, _CHUNK)]

        def issue_inputs(k):
            pltpu.async_copy(x_slice(k), x_v.at[lax.rem(k, _XR)],
                             xsem.at[lax.rem(k, _XR)])
            pltpu.async_copy(proj_hbm.at[idx_all.at[k]],
                             rows_v.at[lax.rem(k, _GR)],
                             gsem.at[lax.rem(k, _GR)])

        for g in range(_CHUNK // _L):
            idn_v[pl.ds(g * _L, _L)] = lax.iota(jnp.int32, _L) + g * _L

        pltpu.sync_copy(t3_hbm.at[wid], idx_all)
        issue_inputs(0)
        issue_inputs(1)

        def chunk_body(k, carry):
            xb = lax.rem(k, _XR)
            gb = lax.rem(k, _GR)
            pltpu.make_async_copy(x_slice(k), x_v.at[xb], xsem.at[xb]).wait()
            pltpu.make_async_copy(proj_hbm.at[idx_all.at[k]],
                                  rows_v.at[gb], gsem.at[gb]).wait()

            pltpu.sync_copy(rows_v.at[gb], x_v.at[xb].at[idn_v], add=True)

            pltpu.async_copy(x_v.at[xb], out_slice(k), osem.at[xb])

            @pl.when(k + 2 < n_chunks)
            def _():
                @pl.when(k >= 1)
                def _():
                    pltpu.make_async_copy(
                        x_v.at[lax.rem(k - 1, _XR)], out_slice(k - 1),
                        osem.at[lax.rem(k - 1, _XR)]
                    ).wait()
                issue_inputs(k + 2)

            return carry

        lax.fori_loop(0, n_chunks, chunk_body, 0)
        for j in range(n_chunks - 3, n_chunks):
            pltpu.make_async_copy(x_v.at[j % _XR], out_slice(j),
                                  osem.at[j % _XR]).wait()

    return run(x, t3, proj)


def kernel(x, t, emb_weight, lin_weight, lin_bias):
    proj = _compute_proj(emb_weight, lin_weight, lin_bias)
    return _sc_gather_add(x, t, proj)


# speedup vs baseline: 4.1799x; 1.9096x over previous
import functools
import jax
import jax.numpy as jnp
from jax import lax
from jax.experimental import pallas as pl
from jax.experimental.pallas import tpu as pltpu
from jax.experimental.pallas import tpu_sc as plsc

_NC = 2
_NS = 16
_NW = _NC * _NS
_L = 16

_CHUNK = 200
_XR = 3


def _proj_body(emb_ref, w_ref, b_ref, out_ref):
    out_ref[...] = lax.dot_general(
        emb_ref[...], w_ref[...],
        (((1,), (1,)), ((), ())),
        preferred_element_type=jnp.float32,
    ) + b_ref[...]


def _compute_proj(emb_weight, lin_weight, lin_bias):
    m, d = emb_weight.shape
    return pl.pallas_call(
        _proj_body,
        out_shape=jax.ShapeDtypeStruct((m, d), jnp.float32),
    )(emb_weight, lin_weight, lin_bias.reshape(1, d))


def _sc_gather_add(x, t, proj):
    n, d = x.shape
    m = proj.shape[0]
    per_w = n // _NW
    assert per_w * _NW == n and per_w % _CHUNK == 0
    n_chunks = per_w // _CHUNK
    t3 = t.reshape(_NW, n_chunks, _CHUNK)
    mesh = plsc.VectorSubcoreMesh(
        core_axis_name="c", subcore_axis_name="s",
        num_cores=_NC, num_subcores=_NS,
    )

    @functools.partial(
        pl.kernel,
        out_type=jax.ShapeDtypeStruct((n, d), jnp.float32),
        mesh=mesh,
        scratch_types=[
            pltpu.VMEM((n_chunks, _CHUNK), jnp.int32),
            pltpu.VMEM((m, d), jnp.float32),
            pltpu.VMEM((_XR, _CHUNK, d), jnp.float32),
            pltpu.SemaphoreType.DMA((_XR,)),
            pltpu.SemaphoreType.DMA((_XR,)),
        ],
    )
    def run(x_hbm, t3_hbm, proj_hbm, out_hbm, idx_all, table_v, x_v,
            xsem, osem):
        wid = lax.axis_index("s") * _NC + lax.axis_index("c")
        w_base = wid * per_w

        def x_slice(k):
            return x_hbm.at[pl.ds(w_base + k * _CHUNK, _CHUNK)]

        def out_slice(k):
            return out_hbm.at[pl.ds(w_base + k * _CHUNK, _CHUNK)]

        def issue_x(k):
            kb = lax.rem(k, _XR)
            pltpu.async_copy(x_slice(k), x_v.at[kb], xsem.at[kb])

        pltpu.sync_copy(proj_hbm, table_v)
        pltpu.sync_copy(t3_hbm.at[wid], idx_all)
        issue_x(0)
        issue_x(1)

        def chunk_body(k, carry):
            xb = lax.rem(k, _XR)
            pltpu.make_async_copy(x_slice(k), x_v.at[xb], xsem.at[xb]).wait()

            def add_tokens(off, j_lo):
                tv = idx_all[k, pl.ds(off, _L)]
                for j in range(j_lo, _L):
                    ti = tv[j]
                    for c in range(d // _L):
                        sl = pl.ds(c * _L, _L)
                        plsc.addupdate(x_v.at[xb, off + j, sl],
                                       table_v[ti, sl])

            @plsc.parallel_loop(0, _CHUNK // _L)
            def tok_body(g):
                add_tokens(g * _L, 0)

            if _CHUNK % _L:
                add_tokens(_CHUNK - _L, _L - _CHUNK % _L)

            pltpu.async_copy(x_v.at[xb], out_slice(k), osem.at[xb])

            @pl.when(k + 2 < n_chunks)
            def _():
                @pl.when(k >= 1)
                def _():
                    pltpu.make_async_copy(
                        x_v.at[lax.rem(k - 1, _XR)], out_slice(k - 1),
                        osem.at[lax.rem(k - 1, _XR)],
                    ).wait()
                issue_x(k + 2)

            return carry

        lax.fori_loop(0, n_chunks, chunk_body, 0)
        for j in range(n_chunks - 3, n_chunks):
            pltpu.make_async_copy(x_v.at[j % _XR], out_slice(j),
                                  osem.at[j % _XR]).wait()

    return run(x, t3, proj)


def kernel(x, t, emb_weight, lin_weight, lin_bias):
    proj = _compute_proj(emb_weight, lin_weight, lin_bias)
    return _sc_gather_add(x, t, proj)
